# Initial kernel scaffold; baseline (speedup 1.0000x reference)
import jax
import jax.numpy as jnp
from jax.experimental import pallas as pl


def kernel(x, edge_index, edges_type, W_rel, W_self, b_pass, W_u1, b_u1, W_u2, b_u2):
    raise NotImplementedError("write your pallas kernel here")


# speedup vs baseline: 9.3665x; 9.3665x over previous
import functools

import jax
import jax.numpy as jnp
from jax import lax
from jax.experimental import pallas as pl
from jax.experimental.pallas import tpu as pltpu
from jax.experimental.pallas import tpu_sc as plsc

N = 10000
E = 320000
D = 128
R = 4

NC = 2
NS = 16
L = 16
NW = NC * NS
EPW = E // NW
CSZ = 128
NCHUNK = EPW // CSZ
REM = EPW - NCHUNK * CSZ
RPS = 624
RTAIL = N - NS * RPS

BN = 400
NB = N // BN


def _build_y_body(x_ref, w_ref, y_ref):
    y_ref[...] = jnp.dot(x_ref[...], w_ref[0], preferred_element_type=jnp.float32)


def _build_y(x, W_rel):
    return pl.pallas_call(
        _build_y_body,
        grid=(R, NB),
        in_specs=[
            pl.BlockSpec((BN, D), lambda r, n: (n, 0)),
            pl.BlockSpec((1, D, D), lambda r, n: (r, 0, 0)),
        ],
        out_specs=pl.BlockSpec((BN, D), lambda r, n: (r * NB + n, 0)),
        out_shape=jax.ShapeDtypeStruct((R * N, D), jnp.float32),
    )(x, W_rel)


_sc_mesh = plsc.VectorSubcoreMesh(
    core_axis_name="c", subcore_axis_name="s", num_cores=NC, num_subcores=NS
)


@functools.partial(
    pl.kernel,
    out_type=jax.ShapeDtypeStruct((NC * N, D), jnp.float32),
    mesh=_sc_mesh,
    scratch_types=[
        pltpu.VMEM((EPW,), jnp.int32),
        pltpu.VMEM((EPW,), jnp.int32),
        pltpu.VMEM((EPW,), jnp.int32),
        pltpu.VMEM((CSZ,), jnp.int32),
        pltpu.VMEM((CSZ,), jnp.int32),
        pltpu.VMEM((CSZ, D), jnp.float32),
        pltpu.VMEM((REM,), jnp.int32),
        pltpu.VMEM((REM,), jnp.int32),
        pltpu.VMEM((REM, D), jnp.float32),
        pltpu.VMEM_SHARED((N, D), jnp.float32),
        pltpu.SemaphoreType.DMA,
    ],
)
def _sc_segsum(src_hbm, dst_hbm, typ_hbm, y_hbm, zeros_hbm, out_hbm,
               src_v, dst_v, typ_v, g_v, d_v, rows_v,
               gr_v, dr_v, rowsr_v, accum, sem):
    cid = lax.axis_index("c")
    sid = lax.axis_index("s")
    wid = sid * NC + cid

    pltpu.sync_copy(zeros_hbm.at[pl.ds(sid * RPS, RPS)],
                    accum.at[pl.ds(sid * RPS, RPS)])
    @pl.when(sid == NS - 1)
    def _():
        pltpu.sync_copy(zeros_hbm.at[pl.ds(NS * RPS, RTAIL)],
                        accum.at[pl.ds(NS * RPS, RTAIL)])

    base = wid * EPW
    pltpu.sync_copy(src_hbm.at[pl.ds(base, EPW)], src_v)
    pltpu.sync_copy(dst_hbm.at[pl.ds(base, EPW)], dst_v)
    pltpu.sync_copy(typ_hbm.at[pl.ds(base, EPW)], typ_v)

    plsc.subcore_barrier()

    def chunk_body(j, _):
        off = pl.multiple_of(j * CSZ, CSZ)
        for k in range(CSZ // L):
            s16 = src_v[pl.ds(off + k * L, L)]
            t16 = typ_v[pl.ds(off + k * L, L)]
            g_v[pl.ds(k * L, L)] = t16 * N + s16
            d_v[pl.ds(k * L, L)] = dst_v[pl.ds(off + k * L, L)]
        pltpu.async_copy(y_hbm.at[g_v], rows_v, sem).wait()
        pltpu.sync_copy(rows_v, accum.at[d_v], add=True)
        return ()

    lax.fori_loop(0, NCHUNK, chunk_body, (), unroll=False)

    roff = NCHUNK * CSZ
    gr_v[...] = typ_v[pl.ds(roff, REM)] * N + src_v[pl.ds(roff, REM)]
    dr_v[...] = dst_v[pl.ds(roff, REM)]
    pltpu.async_copy(y_hbm.at[gr_v], rowsr_v, sem).wait()
    pltpu.sync_copy(rowsr_v, accum.at[dr_v], add=True)

    plsc.subcore_barrier()

    pltpu.sync_copy(accum.at[pl.ds(sid * RPS, RPS)],
                    out_hbm.at[pl.ds(cid * N + sid * RPS, RPS)])
    @pl.when(sid == NS - 1)
    def _():
        pltpu.sync_copy(accum.at[pl.ds(NS * RPS, RTAIL)],
                        out_hbm.at[pl.ds(cid * N + NS * RPS, RTAIL)])


def _mlp_body(x_ref, p0_ref, p1_ref, ws_ref, bp_ref, wu1t_ref, b1_ref,
              wu2t_ref, b2_ref, o_ref):
    x = x_ref[...]
    mid = (p0_ref[...] + p1_ref[...] + bp_ref[...] +
           jnp.dot(x, ws_ref[...], preferred_element_type=jnp.float32))
    h = (jnp.dot(x, wu1t_ref[:D], preferred_element_type=jnp.float32) +
         jnp.dot(mid, wu1t_ref[D:], preferred_element_type=jnp.float32) +
         b1_ref[...])
    h = jnp.maximum(h, 0.0)
    o_ref[...] = jnp.dot(h, wu2t_ref[...], preferred_element_type=jnp.float32) + b2_ref[...]


def _mlp(x, partials, W_self, b_pass, W_u1T, b_u1, W_u2T, b_u2):
    full = lambda a, b: pl.BlockSpec((a, b), lambda n: (0, 0))
    return pl.pallas_call(
        _mlp_body,
        grid=(NB,),
        in_specs=[
            pl.BlockSpec((BN, D), lambda n: (n, 0)),
            pl.BlockSpec((BN, D), lambda n: (n, 0)),
            pl.BlockSpec((BN, D), lambda n: (NB + n, 0)),
            full(D, D),
            full(1, D),
            full(2 * D, 2 * D),
            full(1, 2 * D),
            full(2 * D, D),
            full(1, D),
        ],
        out_specs=pl.BlockSpec((BN, D), lambda n: (n, 0)),
        out_shape=jax.ShapeDtypeStruct((N, D), jnp.float32),
    )(x, partials, partials, W_self, b_pass, W_u1T, b_u1, W_u2T, b_u2)


def kernel(x, edge_index, edges_type, W_rel, W_self, b_pass, W_u1, b_u1, W_u2, b_u2):
    src = edge_index[0]
    dst = edge_index[1]
    Y = _build_y(x, W_rel)
    zeros = jnp.zeros((N, D), dtype=jnp.float32)
    partials = _sc_segsum(src, dst, edges_type, Y, zeros)
    return _mlp(x, partials, W_self, b_pass.reshape(1, D), W_u1.T,
                b_u1.reshape(1, 2 * D), W_u2.T, b_u2.reshape(1, D))


# speedup vs baseline: 12.6485x; 1.3504x over previous
import functools

import jax
import jax.numpy as jnp
from jax import lax
from jax.experimental import pallas as pl
from jax.experimental.pallas import tpu as pltpu
from jax.experimental.pallas import tpu_sc as plsc

N = 10000
E = 320000
D = 128
R = 4

NC = 2
NS = 16
L = 16
NW = NC * NS
EPW = E // NW
CSZ = 128
NCHUNK = EPW // CSZ
NBUF = 6
NHEX = NCHUNK // NBUF
REM = EPW - NCHUNK * CSZ
RPS = 624
RTAIL = N - NS * RPS

BN = 400
NB = N // BN


def _build_y_body(x_ref, w_ref, y_ref):
    y_ref[...] = jnp.dot(x_ref[...], w_ref[0], preferred_element_type=jnp.float32)


def _build_y(x, W_rel):
    return pl.pallas_call(
        _build_y_body,
        grid=(R, NB),
        in_specs=[
            pl.BlockSpec((BN, D), lambda r, n: (n, 0)),
            pl.BlockSpec((1, D, D), lambda r, n: (r, 0, 0)),
        ],
        out_specs=pl.BlockSpec((BN, D), lambda r, n: (r * NB + n, 0)),
        out_shape=jax.ShapeDtypeStruct((R * N, D), jnp.float32),
    )(x, W_rel)


_sc_mesh = plsc.VectorSubcoreMesh(
    core_axis_name="c", subcore_axis_name="s", num_cores=NC, num_subcores=NS
)


@functools.partial(
    pl.kernel,
    out_type=jax.ShapeDtypeStruct((NC * N, D), jnp.float32),
    mesh=_sc_mesh,
    scratch_types=(
        [pltpu.VMEM((CSZ,), jnp.int32) for _ in range(4 * NBUF)]
        + [pltpu.VMEM((CSZ, D), jnp.float32) for _ in range(2)]
        + [pltpu.VMEM((REM,), jnp.int32) for _ in range(4)]
        + [pltpu.VMEM((REM, D), jnp.float32)]
        + [pltpu.VMEM_SHARED((N, D), jnp.float32)]
        + [pltpu.SemaphoreType.DMA for _ in range(NBUF + 2)]
    ),
)
def _sc_segsum(src_hbm, dst_hbm, typ_hbm, y_hbm, out_hbm, *scr):
    sbufs = [scr[4 * i:4 * i + 4] for i in range(NBUF)]
    rows = scr[4 * NBUF:4 * NBUF + 2]
    sR, tR, dR, gR = scr[4 * NBUF + 2:4 * NBUF + 6]
    rowsR = scr[4 * NBUF + 6]
    accum = scr[4 * NBUF + 7]
    ssem = scr[4 * NBUF + 8:4 * NBUF + 8 + NBUF]
    gsem = scr[4 * NBUF + 8 + NBUF:]

    cid = lax.axis_index("c")
    sid = lax.axis_index("s")
    wid = sid * NC + cid
    base = wid * EPW

    def stage_start(k, bi):
        s, t, d, _ = sbufs[bi]
        off = base + k * CSZ
        pltpu.make_async_copy(src_hbm.at[pl.ds(off, CSZ)], s, ssem[bi]).start()
        pltpu.make_async_copy(typ_hbm.at[pl.ds(off, CSZ)], t, ssem[bi]).start()
        pltpu.make_async_copy(dst_hbm.at[pl.ds(off, CSZ)], d, ssem[bi]).start()

    def stage_wait(bi):
        s, t, d, _ = sbufs[bi]
        pltpu.make_async_copy(src_hbm.at[pl.ds(0, CSZ)], s, ssem[bi]).wait()
        pltpu.make_async_copy(src_hbm.at[pl.ds(0, CSZ)], t, ssem[bi]).wait()
        pltpu.make_async_copy(src_hbm.at[pl.ds(0, CSZ)], d, ssem[bi]).wait()

    def fill_g(bi):
        s, t, _, g = sbufs[bi]
        for k in range(CSZ // L):
            g[pl.ds(k * L, L)] = t[pl.ds(k * L, L)] * N + s[pl.ds(k * L, L)]

    def gather_start(bi, p):
        pltpu.make_async_copy(y_hbm.at[sbufs[bi][3]], rows[p], gsem[p]).start()

    def gather_wait_scatter(bi, p):
        pltpu.make_async_copy(y_hbm.at[sbufs[bi][3]], rows[p], gsem[p]).wait()
        pltpu.sync_copy(rows[p], accum.at[sbufs[bi][2]], add=True)

    for i in range(NBUF):
        stage_start(i, i)

    def zfill(i, _):
        for k in range(D // L):
            rows[0][i, pl.ds(k * L, L)] = jnp.zeros((L,), jnp.float32)
        return ()
    lax.fori_loop(0, CSZ, zfill, ())
    for k in range(RPS // CSZ):
        pltpu.sync_copy(rows[0], accum.at[pl.ds(sid * RPS + k * CSZ, CSZ)])
    pltpu.sync_copy(rows[0].at[pl.ds(0, RPS - (RPS // CSZ) * CSZ)],
                    accum.at[pl.ds(sid * RPS + (RPS // CSZ) * CSZ,
                                   RPS - (RPS // CSZ) * CSZ)])
    @pl.when(sid == NS - 1)
    def _():
        pltpu.sync_copy(rows[0].at[pl.ds(0, RTAIL)],
                        accum.at[pl.ds(NS * RPS, RTAIL)])

    plsc.subcore_barrier()

    def hex_body(jj, _):
        e = jj * NBUF
        for p in range(NBUF):
            stage_wait(p)
            fill_g(p)
            gather_start(p, p & 1)
            if p == 0:
                @pl.when(jj > 0)
                def _():
                    gather_wait_scatter(NBUF - 1, (NBUF - 1) & 1)
                    stage_start(e - 1 + NBUF, NBUF - 1)
            else:
                gather_wait_scatter(p - 1, (p - 1) & 1)
                c6 = e + p - 1 + NBUF

                @pl.when(c6 < NCHUNK)
                def _():
                    stage_start(c6, p - 1)
        return ()

    lax.fori_loop(0, NHEX, hex_body, (), unroll=False)

    gather_wait_scatter(NBUF - 1, (NBUF - 1) & 1)

    roff = base + NCHUNK * CSZ
    pltpu.sync_copy(src_hbm.at[pl.ds(roff, REM)], sR)
    pltpu.sync_copy(typ_hbm.at[pl.ds(roff, REM)], tR)
    pltpu.sync_copy(dst_hbm.at[pl.ds(roff, REM)], dR)
    gR[...] = tR[...] * N + sR[...]
    pltpu.async_copy(y_hbm.at[gR], rowsR, gsem[0]).wait()
    pltpu.sync_copy(rowsR, accum.at[dR], add=True)

    plsc.subcore_barrier()

    pltpu.sync_copy(accum.at[pl.ds(sid * RPS, RPS)],
                    out_hbm.at[pl.ds(cid * N + sid * RPS, RPS)])
    @pl.when(sid == NS - 1)
    def _():
        pltpu.sync_copy(accum.at[pl.ds(NS * RPS, RTAIL)],
                        out_hbm.at[pl.ds(cid * N + NS * RPS, RTAIL)])


_CT = (((1,), (1,)), ((), ()))


def _mlp_body(x_ref, p0_ref, p1_ref, ws_ref, bp_ref, wu1_ref, b1_ref,
              wu2_ref, b2_ref, o_ref):
    x = x_ref[...]
    mid = (p0_ref[...] + p1_ref[...] + bp_ref[...] +
           jnp.dot(x, ws_ref[...], preferred_element_type=jnp.float32))
    h = (lax.dot_general(x, wu1_ref[:, :D], _CT, preferred_element_type=jnp.float32) +
         lax.dot_general(mid, wu1_ref[:, D:], _CT, preferred_element_type=jnp.float32) +
         b1_ref[...])
    h = jnp.maximum(h, 0.0)
    o_ref[...] = (lax.dot_general(h, wu2_ref[...], _CT,
                                  preferred_element_type=jnp.float32) + b2_ref[...])


def _mlp(x, partials, W_self, b_pass, W_u1, b_u1, W_u2, b_u2):
    full = lambda a, b: pl.BlockSpec((a, b), lambda n: (0, 0))
    return pl.pallas_call(
        _mlp_body,
        grid=(NB,),
        in_specs=[
            pl.BlockSpec((BN, D), lambda n: (n, 0)),
            pl.BlockSpec((BN, D), lambda n: (n, 0)),
            pl.BlockSpec((BN, D), lambda n: (NB + n, 0)),
            full(D, D),
            full(1, D),
            full(2 * D, 2 * D),
            full(1, 2 * D),
            full(D, 2 * D),
            full(1, D),
        ],
        out_specs=pl.BlockSpec((BN, D), lambda n: (n, 0)),
        out_shape=jax.ShapeDtypeStruct((N, D), jnp.float32),
    )(x, partials, partials, W_self, b_pass, W_u1, b_u1, W_u2, b_u2)


def kernel(x, edge_index, edges_type, W_rel, W_self, b_pass, W_u1, b_u1, W_u2, b_u2):
    Y = _build_y(x, W_rel)
    partials = _sc_segsum(edge_index[0], edge_index[1], edges_type, Y)
    return _mlp(x, partials, W_self, b_pass.reshape(1, D), W_u1,
                b_u1.reshape(1, 2 * D), W_u2, b_u2.reshape(1, D))


# speedup vs baseline: 13.6756x; 1.0812x over previous
import functools

import jax
import jax.numpy as jnp
from jax import lax
from jax.experimental import pallas as pl
from jax.experimental.pallas import tpu as pltpu
from jax.experimental.pallas import tpu_sc as plsc

N = 10000
E = 320000
D = 128
R = 4

NC = 2
NS = 16
L = 16
NW = NC * NS
CSZ = 128
TOTCH = E // CSZ
NFULL = TOTCH // NW
NEXTRA = TOTCH % NW
NBUF = 6
NHEX = NFULL // NBUF
RPS = 624
RTAIL = N - NS * RPS

BN = 400
NB = N // BN


def _build_y_body(x_ref, w_ref, y_ref):
    y_ref[...] = jnp.dot(x_ref[...], w_ref[0], preferred_element_type=jnp.float32)


def _build_y(x, W_rel):
    return pl.pallas_call(
        _build_y_body,
        grid=(NB, R),
        in_specs=[
            pl.BlockSpec((BN, D), lambda n, r: (n, 0)),
            pl.BlockSpec((1, D, D), lambda n, r: (r, 0, 0)),
        ],
        out_specs=pl.BlockSpec((BN, D), lambda n, r: (r * NB + n, 0)),
        out_shape=jax.ShapeDtypeStruct((R * N, D), jnp.float32),
    )(x, W_rel)


_sc_mesh = plsc.VectorSubcoreMesh(
    core_axis_name="c", subcore_axis_name="s", num_cores=NC, num_subcores=NS
)


@functools.partial(
    pl.kernel,
    out_type=jax.ShapeDtypeStruct((NC * N, D), jnp.float32),
    mesh=_sc_mesh,
    scratch_types=(
        [pltpu.VMEM((2, CSZ), jnp.int32) for _ in range(NBUF)]
        + [pltpu.VMEM((CSZ,), jnp.int32) for _ in range(2 * NBUF)]
        + [pltpu.VMEM((CSZ, D), jnp.float32) for _ in range(2)]
        + [pltpu.VMEM_SHARED((N, D), jnp.float32)]
        + [pltpu.SemaphoreType.DMA for _ in range(NBUF + 2)]
    ),
)
def _sc_segsum(ei_hbm, typ_hbm, y_hbm, out_hbm, *scr):
    ei = scr[0:NBUF]
    typ = scr[NBUF:2 * NBUF]
    gidx = scr[2 * NBUF:3 * NBUF]
    rows = scr[3 * NBUF:3 * NBUF + 2]
    accum = scr[3 * NBUF + 2]
    ssem = scr[3 * NBUF + 3:3 * NBUF + 3 + NBUF]
    gsem = scr[3 * NBUF + 3 + NBUF:]

    cid = lax.axis_index("c")
    sid = lax.axis_index("s")
    wid = sid * NC + cid

    def stage_start(i, bi):
        off = (wid + NW * i) * CSZ
        pltpu.make_async_copy(ei_hbm.at[:, pl.ds(off, CSZ)], ei[bi], ssem[bi]).start()
        pltpu.make_async_copy(typ_hbm.at[pl.ds(off, CSZ)], typ[bi], ssem[bi]).start()

    def stage_wait(bi):
        pltpu.make_async_copy(ei_hbm.at[:, pl.ds(0, CSZ)], ei[bi], ssem[bi]).wait()
        pltpu.make_async_copy(typ_hbm.at[pl.ds(0, CSZ)], typ[bi], ssem[bi]).wait()

    def fill_g(bi):
        for k in range(CSZ // L):
            gidx[bi][pl.ds(k * L, L)] = (
                typ[bi][pl.ds(k * L, L)] * N + ei[bi][0, pl.ds(k * L, L)])

    def gather_start(bi, p):
        pltpu.make_async_copy(y_hbm.at[gidx[bi]], rows[p], gsem[p]).start()

    def gather_wait_scatter(bi, p):
        pltpu.make_async_copy(y_hbm.at[gidx[bi]], rows[p], gsem[p]).wait()
        pltpu.sync_copy(rows[p], accum.at[ei[bi].at[1]], add=True)

    for i in range(NBUF):
        stage_start(i, i)

    def zfill(i, _):
        for k in range(D // L):
            rows[0][i, pl.ds(k * L, L)] = jnp.zeros((L,), jnp.float32)
        return ()
    lax.fori_loop(0, CSZ, zfill, ())
    for k in range(RPS // CSZ):
        pltpu.sync_copy(rows[0], accum.at[pl.ds(sid * RPS + k * CSZ, CSZ)])
    pltpu.sync_copy(rows[0].at[pl.ds(0, RPS - (RPS // CSZ) * CSZ)],
                    accum.at[pl.ds(sid * RPS + (RPS // CSZ) * CSZ,
                                   RPS - (RPS // CSZ) * CSZ)])
    @pl.when(sid == NS - 1)
    def _():
        pltpu.sync_copy(rows[0].at[pl.ds(0, RTAIL)],
                        accum.at[pl.ds(NS * RPS, RTAIL)])

    plsc.subcore_barrier()

    def hex_body(jj, _):
        e = jj * NBUF
        for p in range(NBUF):
            stage_wait(p)
            fill_g(p)
            gather_start(p, p & 1)
            if p == 0:
                @pl.when(jj > 0)
                def _():
                    gather_wait_scatter(NBUF - 1, (NBUF - 1) & 1)
                    stage_start(e - 1 + NBUF, NBUF - 1)
            else:
                gather_wait_scatter(p - 1, (p - 1) & 1)
                c6 = e + p - 1 + NBUF

                @pl.when(c6 < NFULL)
                def _():
                    stage_start(c6, p - 1)
        return ()

    lax.fori_loop(0, NHEX, hex_body, (), unroll=False)

    gather_wait_scatter(NBUF - 1, (NBUF - 1) & 1)

    @pl.when(wid < NEXTRA)
    def _():
        stage_start(NFULL, 0)
        stage_wait(0)
        fill_g(0)
        gather_start(0, 0)
        gather_wait_scatter(0, 0)

    plsc.subcore_barrier()

    pltpu.sync_copy(accum.at[pl.ds(sid * RPS, RPS)],
                    out_hbm.at[pl.ds(cid * N + sid * RPS, RPS)])
    @pl.when(sid == NS - 1)
    def _():
        pltpu.sync_copy(accum.at[pl.ds(NS * RPS, RTAIL)],
                        out_hbm.at[pl.ds(cid * N + NS * RPS, RTAIL)])


_CT = (((1,), (1,)), ((), ()))


def _mlp_body(x_ref, p0_ref, p1_ref, ws_ref, bp_ref, wu1_ref, b1_ref,
              wu2_ref, b2_ref, o_ref):
    x = x_ref[...]
    mid = (p0_ref[...] + p1_ref[...] + bp_ref[...] +
           jnp.dot(x, ws_ref[...], preferred_element_type=jnp.float32))
    h = (lax.dot_general(x, wu1_ref[:, :D], _CT, preferred_element_type=jnp.float32) +
         lax.dot_general(mid, wu1_ref[:, D:], _CT, preferred_element_type=jnp.float32) +
         b1_ref[...])
    h = jnp.maximum(h, 0.0)
    o_ref[...] = (lax.dot_general(h, wu2_ref[...], _CT,
                                  preferred_element_type=jnp.float32) + b2_ref[...])


def _mlp(x, partials, W_self, b_pass, W_u1, b_u1, W_u2, b_u2):
    full = lambda a, b: pl.BlockSpec((a, b), lambda n: (0, 0))
    return pl.pallas_call(
        _mlp_body,
        grid=(NB,),
        in_specs=[
            pl.BlockSpec((BN, D), lambda n: (n, 0)),
            pl.BlockSpec((BN, D), lambda n: (n, 0)),
            pl.BlockSpec((BN, D), lambda n: (NB + n, 0)),
            full(D, D),
            full(1, D),
            full(2 * D, 2 * D),
            full(1, 2 * D),
            full(D, 2 * D),
            full(1, D),
        ],
        out_specs=pl.BlockSpec((BN, D), lambda n: (n, 0)),
        out_shape=jax.ShapeDtypeStruct((N, D), jnp.float32),
    )(x, partials, partials, W_self, b_pass, W_u1, b_u1, W_u2, b_u2)


def kernel(x, edge_index, edges_type, W_rel, W_self, b_pass, W_u1, b_u1, W_u2, b_u2):
    Y = _build_y(x, W_rel)
    partials = _sc_segsum(edge_index, edges_type, Y)
    return _mlp(x, partials, W_self, b_pass.reshape(1, D), W_u1,
                b_u1.reshape(1, 2 * D), W_u2, b_u2.reshape(1, D))


# speedup vs baseline: 19.9235x; 1.4569x over previous
import functools

import jax
import jax.numpy as jnp
from jax import lax
from jax.experimental import pallas as pl
from jax.experimental.pallas import tpu as pltpu
from jax.experimental.pallas import tpu_sc as plsc

N = 10000
E = 320000
D = 128
R = 4

NC = 2
NS = 16
L = 16
NW = NC * NS
CSZ = 128
TOTCH = E // CSZ
NFULL = TOTCH // NW
NEXTRA = TOTCH % NW
NDST = 6
NST = 3
NROW = 3
NHEX = NFULL // NDST
RPS = 624
RTAIL = N - NS * RPS

BN = 1000
NB = N // BN


def _build_y_body(x_ref, w_ref, y_ref):
    x = x_ref[...]
    for r in range(R):
        y_ref[r] = jnp.dot(x, w_ref[r], preferred_element_type=jnp.float32)


def _build_y(x, W_rel):
    return pl.pallas_call(
        _build_y_body,
        grid=(NB,),
        in_specs=[
            pl.BlockSpec((BN, D), lambda n: (n, 0)),
            pl.BlockSpec((R, D, D), lambda n: (0, 0, 0)),
        ],
        out_specs=pl.BlockSpec((R, BN, D), lambda n: (0, n, 0)),
        out_shape=jax.ShapeDtypeStruct((R, N, D), jnp.float32),
    )(x, W_rel)


_sc_mesh = plsc.VectorSubcoreMesh(
    core_axis_name="c", subcore_axis_name="s", num_cores=NC, num_subcores=NS
)


@functools.partial(
    pl.kernel,
    out_type=jax.ShapeDtypeStruct((NC * N, D), jnp.float32),
    mesh=_sc_mesh,
    scratch_types=(
        [pltpu.VMEM((CSZ,), jnp.int32) for _ in range(NDST)]
        + [pltpu.VMEM((CSZ,), jnp.int32) for _ in range(NST)]
        + [pltpu.VMEM((CSZ,), jnp.int32) for _ in range(NST)]
        + [pltpu.VMEM((CSZ, D), jnp.float32) for _ in range(NROW)]
        + [pltpu.VMEM_SHARED((N, D), jnp.float32)]
        + [pltpu.SemaphoreType.DMA for _ in range(NDST)]
        + [pltpu.SemaphoreType.DMA for _ in range(NST)]
        + [pltpu.SemaphoreType.DMA for _ in range(NROW)]
        + [pltpu.SemaphoreType.DMA]
    ),
)
def _sc_segsum(ei_hbm, typ_hbm, y_hbm, out_hbm, *scr):
    dstb = scr[0:NDST]
    srcb = scr[NDST:NDST + NST]
    gb = scr[NDST + NST:NDST + 2 * NST]
    rows = scr[NDST + 2 * NST:NDST + 2 * NST + NROW]
    accum = scr[NDST + 2 * NST + NROW]
    o = NDST + 2 * NST + NROW + 1
    dsem = scr[o:o + NDST]
    stsem = scr[o + NDST:o + NDST + NST]
    gsem = scr[o + NDST + NST:o + NDST + NST + NROW]
    scsem = scr[o + NDST + NST + NROW]

    cid = lax.axis_index("c")
    sid = lax.axis_index("s")
    wid = sid * NC + cid

    def stage_dst(i, bi):
        off = E + (wid + NW * i) * CSZ
        pltpu.make_async_copy(ei_hbm.at[pl.ds(off, CSZ)], dstb[bi], dsem[bi]).start()

    def stage_st(i, bi):
        off = (wid + NW * i) * CSZ
        pltpu.make_async_copy(ei_hbm.at[pl.ds(off, CSZ)], srcb[bi], stsem[bi]).start()
        pltpu.make_async_copy(typ_hbm.at[pl.ds(off, CSZ)], gb[bi], stsem[bi]).start()

    def wait_dst(bi):
        pltpu.make_async_copy(ei_hbm.at[pl.ds(0, CSZ)], dstb[bi], dsem[bi]).wait()

    def wait_st(bi):
        pltpu.make_async_copy(ei_hbm.at[pl.ds(0, CSZ)], srcb[bi], stsem[bi]).wait()
        pltpu.make_async_copy(ei_hbm.at[pl.ds(0, CSZ)], gb[bi], stsem[bi]).wait()

    def fill_g(bi):
        for k in range(CSZ // L):
            gb[bi][pl.ds(k * L, L)] = (
                gb[bi][pl.ds(k * L, L)] * N + srcb[bi][pl.ds(k * L, L)])

    def gather_start(sbi, p):
        pltpu.make_async_copy(y_hbm.at[gb[sbi]], rows[p], gsem[p]).start()

    def gather_wait(sbi, p):
        pltpu.make_async_copy(y_hbm.at[gb[sbi]], rows[p], gsem[p]).wait()

    def scatter_start(p, dbi):
        pltpu.make_async_copy(rows[p], accum.at[dstb[dbi]], scsem).start(add=True)

    def scatter_wait(p, dbi):
        pltpu.make_async_copy(rows[p], accum.at[dstb[dbi]], scsem).wait()

    for i in range(NDST):
        stage_dst(i, i)
    for i in range(NST):
        stage_st(i, i)

    def zfill(i, _):
        for k in range(D // L):
            rows[0][i, pl.ds(k * L, L)] = jnp.zeros((L,), jnp.float32)
        return ()
    lax.fori_loop(0, CSZ, zfill, ())
    for k in range(RPS // CSZ):
        pltpu.sync_copy(rows[0], accum.at[pl.ds(sid * RPS + k * CSZ, CSZ)])
    pltpu.sync_copy(rows[0].at[pl.ds(0, RPS - (RPS // CSZ) * CSZ)],
                    accum.at[pl.ds(sid * RPS + (RPS // CSZ) * CSZ,
                                   RPS - (RPS // CSZ) * CSZ)])
    @pl.when(sid == NS - 1)
    def _():
        pltpu.sync_copy(rows[0].at[pl.ds(0, RTAIL)],
                        accum.at[pl.ds(NS * RPS, RTAIL)])

    plsc.subcore_barrier()

    def hex_body(jj, _):
        e = jj * NDST
        for p in range(NDST):
            c = e + p
            wait_st(p % NST)
            wait_dst(p)
            fill_g(p % NST)

            def retire3():
                scatter_wait(p % NROW, (p + 3) % NDST)
                stage_dst(c + 3, (p + 3) % NDST)

            def launch2():
                gather_wait((p + 1) % NST, (p + 1) % NROW)
                scatter_start((p + 1) % NROW, (p + 4) % NDST)

                @pl.when(c + 1 < NFULL)
                def _():
                    stage_st(c + 1, (p + 1) % NST)

            if p >= 3:
                scatter_wait(p % NROW, (p + 3) % NDST)

                @pl.when(c + 3 < NFULL)
                def _():
                    stage_dst(c + 3, (p + 3) % NDST)
            else:
                @pl.when(jj > 0)
                def _():
                    retire3()

            if p >= 2:
                launch2()
            else:
                @pl.when(jj > 0)
                def _():
                    launch2()

            gather_start(p % NST, p % NROW)
        return ()

    lax.fori_loop(0, NHEX, hex_body, (), unroll=False)

    scatter_wait((NFULL - 3) % NROW, (NFULL - 3) % NDST)
    gather_wait((NFULL - 2) % NST, (NFULL - 2) % NROW)
    scatter_start((NFULL - 2) % NROW, (NFULL - 2) % NDST)
    scatter_wait((NFULL - 2) % NROW, (NFULL - 2) % NDST)
    gather_wait((NFULL - 1) % NST, (NFULL - 1) % NROW)
    scatter_start((NFULL - 1) % NROW, (NFULL - 1) % NDST)
    scatter_wait((NFULL - 1) % NROW, (NFULL - 1) % NDST)

    @pl.when(wid < NEXTRA)
    def _():
        stage_dst(NFULL, 0)
        stage_st(NFULL, 0)
        wait_st(0)
        wait_dst(0)
        fill_g(0)
        gather_start(0, 0)
        gather_wait(0, 0)
        scatter_start(0, 0)
        scatter_wait(0, 0)

    plsc.subcore_barrier()

    pltpu.sync_copy(accum.at[pl.ds(sid * RPS, RPS)],
                    out_hbm.at[pl.ds(cid * N + sid * RPS, RPS)])
    @pl.when(sid == NS - 1)
    def _():
        pltpu.sync_copy(accum.at[pl.ds(NS * RPS, RTAIL)],
                        out_hbm.at[pl.ds(cid * N + NS * RPS, RTAIL)])


_CT = (((1,), (1,)), ((), ()))


def _mlp_body(x_ref, p0_ref, p1_ref, ws_ref, bp_ref, wu1_ref, b1_ref,
              wu2_ref, b2_ref, o_ref):
    x = x_ref[...]
    mid = (p0_ref[...] + p1_ref[...] + bp_ref[...] +
           jnp.dot(x, ws_ref[...], preferred_element_type=jnp.float32))
    h = (lax.dot_general(x, wu1_ref[:, :D], _CT, preferred_element_type=jnp.float32) +
         lax.dot_general(mid, wu1_ref[:, D:], _CT, preferred_element_type=jnp.float32) +
         b1_ref[...])
    h = jnp.maximum(h, 0.0)
    o_ref[...] = (lax.dot_general(h, wu2_ref[...], _CT,
                                  preferred_element_type=jnp.float32) + b2_ref[...])


def _mlp(x, partials, W_self, b_pass, W_u1, b_u1, W_u2, b_u2):
    full = lambda a, b: pl.BlockSpec((a, b), lambda n: (0, 0))
    return pl.pallas_call(
        _mlp_body,
        grid=(NB,),
        in_specs=[
            pl.BlockSpec((BN, D), lambda n: (n, 0)),
            pl.BlockSpec((BN, D), lambda n: (n, 0)),
            pl.BlockSpec((BN, D), lambda n: (NB + n, 0)),
            full(D, D),
            full(1, D),
            full(2 * D, 2 * D),
            full(1, 2 * D),
            full(D, 2 * D),
            full(1, D),
        ],
        out_specs=pl.BlockSpec((BN, D), lambda n: (n, 0)),
        out_shape=jax.ShapeDtypeStruct((N, D), jnp.float32),
    )(x, partials, partials, W_self, b_pass, W_u1, b_u1, W_u2, b_u2)


def kernel(x, edge_index, edges_type, W_rel, W_self, b_pass, W_u1, b_u1, W_u2, b_u2):
    Y = _build_y(x, W_rel).reshape(R * N, D)
    partials = _sc_segsum(edge_index.reshape(2 * E), edges_type, Y)
    return _mlp(x, partials, W_self, b_pass.reshape(1, D), W_u1,
                b_u1.reshape(1, 2 * D), W_u2, b_u2.reshape(1, D))


# speedup vs baseline: 19.9812x; 1.0029x over previous
import functools

import jax
import jax.numpy as jnp
from jax import lax
from jax.experimental import pallas as pl
from jax.experimental.pallas import tpu as pltpu
from jax.experimental.pallas import tpu_sc as plsc

N = 10000
E = 320000
D = 128
R = 4

NC = 2
NS = 16
L = 16
NW = NC * NS
CSZ = 128
TOTCH = E // CSZ
NFULL = TOTCH // NW
NEXTRA = TOTCH % NW
NDST = 6
NST = 3
NROW = 3
NHEX = NFULL // NDST
RPS = 624
RTAIL = N - NS * RPS

BN = 1000
NB = N // BN


def _build_y_body(x_ref, w_ref, y_ref):
    x = x_ref[...]
    for r in range(R):
        y_ref[r] = jnp.dot(x, w_ref[r], preferred_element_type=jnp.float32)


def _build_y(x, W_rel):
    return pl.pallas_call(
        _build_y_body,
        grid=(NB,),
        in_specs=[
            pl.BlockSpec((BN, D), lambda n: (n, 0)),
            pl.BlockSpec((R, D, D), lambda n: (0, 0, 0)),
        ],
        out_specs=pl.BlockSpec((R, BN, D), lambda n: (0, n, 0)),
        out_shape=jax.ShapeDtypeStruct((R, N, D), jnp.float32),
    )(x, W_rel)


_sc_mesh = plsc.VectorSubcoreMesh(
    core_axis_name="c", subcore_axis_name="s", num_cores=NC, num_subcores=NS
)


@functools.partial(
    pl.kernel,
    out_type=jax.ShapeDtypeStruct((NC * N, D), jnp.float32),
    mesh=_sc_mesh,
    scratch_types=(
        [pltpu.VMEM((CSZ,), jnp.int32) for _ in range(NDST)]
        + [pltpu.VMEM((CSZ,), jnp.int32) for _ in range(NST)]
        + [pltpu.VMEM((CSZ,), jnp.int32) for _ in range(NST)]
        + [pltpu.VMEM((CSZ, D), jnp.float32) for _ in range(NROW)]
        + [pltpu.VMEM_SHARED((N, D), jnp.float32)]
        + [pltpu.SemaphoreType.DMA for _ in range(NDST)]
        + [pltpu.SemaphoreType.DMA for _ in range(NST)]
        + [pltpu.SemaphoreType.DMA for _ in range(NROW)]
        + [pltpu.SemaphoreType.DMA]
    ),
)
def _sc_segsum(ei_hbm, typ_hbm, y_hbm, out_hbm, *scr):
    dstb = scr[0:NDST]
    srcb = scr[NDST:NDST + NST]
    gb = scr[NDST + NST:NDST + 2 * NST]
    rows = scr[NDST + 2 * NST:NDST + 2 * NST + NROW]
    accum = scr[NDST + 2 * NST + NROW]
    o = NDST + 2 * NST + NROW + 1
    dsem = scr[o:o + NDST]
    stsem = scr[o + NDST:o + NDST + NST]
    gsem = scr[o + NDST + NST:o + NDST + NST + NROW]
    scsem = scr[o + NDST + NST + NROW]

    cid = lax.axis_index("c")
    sid = lax.axis_index("s")
    wid = sid * NC + cid

    def stage_dst(i, bi):
        off = E + (wid + NW * i) * CSZ
        pltpu.make_async_copy(ei_hbm.at[pl.ds(off, CSZ)], dstb[bi], dsem[bi]).start()

    def stage_st(i, bi):
        off = (wid + NW * i) * CSZ
        pltpu.make_async_copy(ei_hbm.at[pl.ds(off, CSZ)], srcb[bi], stsem[bi]).start()
        pltpu.make_async_copy(typ_hbm.at[pl.ds(off, CSZ)], gb[bi], stsem[bi]).start()

    def wait_dst(bi):
        pltpu.make_async_copy(ei_hbm.at[pl.ds(0, CSZ)], dstb[bi], dsem[bi]).wait()

    def wait_st(bi):
        pltpu.make_async_copy(ei_hbm.at[pl.ds(0, CSZ)], srcb[bi], stsem[bi]).wait()
        pltpu.make_async_copy(ei_hbm.at[pl.ds(0, CSZ)], gb[bi], stsem[bi]).wait()

    def fill_g(bi):
        for k in range(CSZ // L):
            gb[bi][pl.ds(k * L, L)] = (
                gb[bi][pl.ds(k * L, L)] * N + srcb[bi][pl.ds(k * L, L)])

    def gather_start(sbi, p):
        pltpu.make_async_copy(y_hbm.at[gb[sbi]], rows[p], gsem[p]).start()

    def gather_wait(sbi, p):
        pltpu.make_async_copy(y_hbm.at[gb[sbi]], rows[p], gsem[p]).wait()

    def scatter_start(p, dbi):
        pltpu.make_async_copy(rows[p], accum.at[dstb[dbi]], scsem).start(add=True)

    def scatter_wait(p, dbi):
        pltpu.make_async_copy(rows[p], accum.at[dstb[dbi]], scsem).wait()

    for i in range(NDST):
        stage_dst(i, i)
    for i in range(NST):
        stage_st(i, i)

    def zfill(i, _):
        for k in range(D // L):
            rows[0][i, pl.ds(k * L, L)] = jnp.zeros((L,), jnp.float32)
        return ()
    lax.fori_loop(0, CSZ, zfill, ())
    for k in range(RPS // CSZ):
        pltpu.sync_copy(rows[0], accum.at[pl.ds(sid * RPS + k * CSZ, CSZ)])
    pltpu.sync_copy(rows[0].at[pl.ds(0, RPS - (RPS // CSZ) * CSZ)],
                    accum.at[pl.ds(sid * RPS + (RPS // CSZ) * CSZ,
                                   RPS - (RPS // CSZ) * CSZ)])
    @pl.when(sid == NS - 1)
    def _():
        pltpu.sync_copy(rows[0].at[pl.ds(0, RTAIL)],
                        accum.at[pl.ds(NS * RPS, RTAIL)])

    plsc.subcore_barrier()

    def hex_body(jj, _):
        e = jj * NDST
        for p in range(NDST):
            c = e + p
            wait_st(p % NST)
            wait_dst(p)
            fill_g(p % NST)

            def retire3():
                scatter_wait(p % NROW, (p + 3) % NDST)
                stage_dst(c + 3, (p + 3) % NDST)

            def launch2():
                gather_wait((p + 1) % NST, (p + 1) % NROW)
                scatter_start((p + 1) % NROW, (p + 4) % NDST)

                @pl.when(c + 1 < NFULL)
                def _():
                    stage_st(c + 1, (p + 1) % NST)

            if p >= 3:
                scatter_wait(p % NROW, (p + 3) % NDST)

                @pl.when(c + 3 < NFULL)
                def _():
                    stage_dst(c + 3, (p + 3) % NDST)
            else:
                @pl.when(jj > 0)
                def _():
                    retire3()

            if p >= 2:
                launch2()
            else:
                @pl.when(jj > 0)
                def _():
                    launch2()

            gather_start(p % NST, p % NROW)
        return ()

    lax.fori_loop(0, NHEX, hex_body, (), unroll=False)

    @pl.when(wid < NEXTRA)
    def _():
        stage_dst(NFULL, 0)
        stage_st(NFULL, 0)

    scatter_wait((NFULL - 3) % NROW, (NFULL - 3) % NDST)
    gather_wait((NFULL - 2) % NST, (NFULL - 2) % NROW)
    scatter_start((NFULL - 2) % NROW, (NFULL - 2) % NDST)
    scatter_wait((NFULL - 2) % NROW, (NFULL - 2) % NDST)
    gather_wait((NFULL - 1) % NST, (NFULL - 1) % NROW)
    scatter_start((NFULL - 1) % NROW, (NFULL - 1) % NDST)
    scatter_wait((NFULL - 1) % NROW, (NFULL - 1) % NDST)

    @pl.when(wid < NEXTRA)
    def _():
        wait_st(0)
        wait_dst(0)
        fill_g(0)
        gather_start(0, 0)
        gather_wait(0, 0)
        scatter_start(0, 0)
        scatter_wait(0, 0)

    plsc.subcore_barrier()

    pltpu.sync_copy(accum.at[pl.ds(sid * RPS, RPS)],
                    out_hbm.at[pl.ds(cid * N + sid * RPS, RPS)])
    @pl.when(sid == NS - 1)
    def _():
        pltpu.sync_copy(accum.at[pl.ds(NS * RPS, RTAIL)],
                        out_hbm.at[pl.ds(cid * N + NS * RPS, RTAIL)])


_CT = (((1,), (1,)), ((), ()))


def _mlp_body(x_ref, p0_ref, p1_ref, ws_ref, bp_ref, wu1_ref, b1_ref,
              wu2_ref, b2_ref, o_ref):
    x = x_ref[...]
    mid = (p0_ref[...] + p1_ref[...] + bp_ref[...] +
           jnp.dot(x, ws_ref[...], preferred_element_type=jnp.float32))
    h = (lax.dot_general(x, wu1_ref[:, :D], _CT, preferred_element_type=jnp.float32) +
         lax.dot_general(mid, wu1_ref[:, D:], _CT, preferred_element_type=jnp.float32) +
         b1_ref[...])
    h = jnp.maximum(h, 0.0)
    o_ref[...] = (lax.dot_general(h, wu2_ref[...], _CT,
                                  preferred_element_type=jnp.float32) + b2_ref[...])


def _mlp(x, partials, W_self, b_pass, W_u1, b_u1, W_u2, b_u2):
    full = lambda a, b: pl.BlockSpec((a, b), lambda n: (0, 0))
    return pl.pallas_call(
        _mlp_body,
        grid=(NB,),
        in_specs=[
            pl.BlockSpec((BN, D), lambda n: (n, 0)),
            pl.BlockSpec((BN, D), lambda n: (n, 0)),
            pl.BlockSpec((BN, D), lambda n: (NB + n, 0)),
            full(D, D),
            full(1, D),
            full(2 * D, 2 * D),
            full(1, 2 * D),
            full(D, 2 * D),
            full(1, D),
        ],
        out_specs=pl.BlockSpec((BN, D), lambda n: (n, 0)),
        out_shape=jax.ShapeDtypeStruct((N, D), jnp.float32),
    )(x, partials, partials, W_self, b_pass, W_u1, b_u1, W_u2, b_u2)


def kernel(x, edge_index, edges_type, W_rel, W_self, b_pass, W_u1, b_u1, W_u2, b_u2):
    Y = _build_y(x, W_rel).reshape(R * N, D)
    partials = _sc_segsum(edge_index.reshape(2 * E), edges_type, Y)
    return _mlp(x, partials, W_self, b_pass.reshape(1, D), W_u1,
                b_u1.reshape(1, 2 * D), W_u2, b_u2.reshape(1, D))
